# baseline (speedup 1.0000x reference)
import jax
import jax.numpy as jnp
from jax.experimental import pallas as pl
from jax.experimental.pallas import tpu as pltpu

D = 512
B = 64
T = 3
N = 50000
BLK = 2000
NB = N // BLK
NEG = -1e30
W = 16


def _split(a):
    hi = a.astype(jnp.bfloat16)
    lo = (a - hi.astype(jnp.float32)).astype(jnp.bfloat16)
    return hi, lo


def _update(w, wsz, seg, xh, xl, h_ref, m_ref, s_ref, u_ref):
    hw = h_ref[pl.ds(w, wsz), :]
    hwh, hwl = _split(hw)

    def dxt(u, v):
        return jax.lax.dot_general(u, v, (((1,), (1,)), ((), ())),
                                   preferred_element_type=jnp.float32)

    z = dxt(jnp.concatenate([hwh, hwl], axis=0), xh)
    e = z[:wsz] + z[wsz:] + dxt(hwh, xl)
    mask = (seg - w) == jax.lax.broadcasted_iota(jnp.int32, (wsz, BLK), 0)
    e_m = jnp.where(mask, e, NEG)
    m_old = m_ref[pl.ds(w, wsz), :]
    m_new = jnp.maximum(m_old, jnp.max(e_m, axis=1, keepdims=True))
    p = jnp.where(mask, jnp.exp(e - m_new), 0.0)
    scale = jnp.exp(m_old - m_new)
    s_ref[pl.ds(w, wsz), :] = (s_ref[pl.ds(w, wsz), :] * scale
                               + jnp.sum(p, axis=1, keepdims=True))
    ph, plo = _split(p)

    def dp(u, v):
        return jax.lax.dot_general(u, v, (((1,), (0,)), ((), ())),
                                   preferred_element_type=jnp.float32)

    y = dp(jnp.concatenate([ph, plo], axis=0), xh)
    u_ref[pl.ds(w, wsz), :] = (u_ref[pl.ds(w, wsz), :] * scale
                               + y[:wsz] + y[wsz:])
    m_ref[pl.ds(w, wsz), :] = m_new


def _softmax_blk(i, win_ref, seg, xh, xl, h_ref, m_ref, s_ref, u_ref):
    w8 = win_ref[i, 0]
    fast = win_ref[i, 1] == 1

    @pl.when(fast)
    def _fast():
        w = pl.multiple_of(w8 * 8, 8)
        _update(w, W, seg, xh, xl, h_ref, m_ref, s_ref, u_ref)

    @pl.when(jnp.logical_not(fast))
    def _general():
        _update(0, B, seg, xh, xl, h_ref, m_ref, s_ref, u_ref)


def _body_a(win_ref, batch_ref, x_ref, b_ref,
            xh_ref, xl_ref, st_ref,
            h_ref, m_ref, s_ref, u_ref):
    i = pl.program_id(0)

    @pl.when(i == 0)
    def _init():
        gates = jnp.broadcast_to(b_ref[...], (B, 4 * D))
        gi = jax.nn.sigmoid(gates[:, 0 * D:1 * D])
        gf = jax.nn.sigmoid(gates[:, 1 * D:2 * D])
        gg = jnp.tanh(gates[:, 2 * D:3 * D])
        go = jax.nn.sigmoid(gates[:, 3 * D:4 * D])
        c_new = gi * gg
        h_new = go * jnp.tanh(c_new)
        h_ref[...] = h_new
        st_ref[B:2 * B, :] = c_new
        m_ref[...] = jnp.full((B, 1), NEG, jnp.float32)
        s_ref[...] = jnp.zeros((B, 1), jnp.float32)
        u_ref[...] = jnp.zeros((B, D), jnp.float32)

    xh, xl = _split(x_ref[...])
    xh_ref[...] = xh
    xl_ref[...] = xl
    seg = batch_ref[0]
    _softmax_blk(i, win_ref, seg, xh, xl, h_ref, m_ref, s_ref, u_ref)

    @pl.when(i == NB - 1)
    def _finalize():
        st_ref[0:B, :] = h_ref[...]
        st_ref[2 * B:3 * B, :] = u_ref[...] / (s_ref[...] + 1e-16)


def _body_b(win_ref, batch_ref, xh_ref, xl_ref,
            wqh_ref, wql_ref, wrh_ref, wrl_ref, b_ref, st_ref,
            out_ref,
            h_ref, c_ref, r_ref, m_ref, s_ref, u_ref):
    t = pl.program_id(0)
    i = pl.program_id(1)

    @pl.when(i == 0)
    def _lstm_and_init():
        first = (t == 0)
        h_prev = jnp.where(first, st_ref[0:B, :], h_ref[...])
        c_prev = jnp.where(first, st_ref[B:2 * B, :], c_ref[...])
        r_prev = jnp.where(first, st_ref[2 * B:3 * B, :], r_ref[...])

        def d(u, v):
            return jax.lax.dot_general(u, v, (((1,), (0,)), ((), ())),
                                       preferred_element_type=jnp.float32)

        hh_, hl_ = _split(h_prev)
        rh_, rl_ = _split(r_prev)
        gates = (d(hh_, wqh_ref[...]) + d(hh_, wql_ref[...]) + d(hl_, wqh_ref[...])
                 + d(rh_, wrh_ref[...]) + d(rh_, wrl_ref[...]) + d(rl_, wrh_ref[...])
                 + b_ref[...])
        gi = jax.nn.sigmoid(gates[:, 0 * D:1 * D])
        gf = jax.nn.sigmoid(gates[:, 1 * D:2 * D])
        gg = jnp.tanh(gates[:, 2 * D:3 * D])
        go = jax.nn.sigmoid(gates[:, 3 * D:4 * D])
        c_new = gf * c_prev + gi * gg
        h_new = go * jnp.tanh(c_new)
        h_ref[...] = h_new
        c_ref[...] = c_new
        m_ref[...] = jnp.full((B, 1), NEG, jnp.float32)
        s_ref[...] = jnp.zeros((B, 1), jnp.float32)
        u_ref[...] = jnp.zeros((B, D), jnp.float32)

    seg = batch_ref[0]
    _softmax_blk(i, win_ref, seg, xh_ref[...], xl_ref[...],
                 h_ref, m_ref, s_ref, u_ref)

    @pl.when(i == NB - 1)
    def _finalize():
        r = u_ref[...] / (s_ref[...] + 1e-16)
        r_ref[...] = r

        @pl.when(t == T - 2)
        def _write_out():
            out_ref[:, :D] = h_ref[...]
            out_ref[:, D:] = r


def kernel(x, batch, W_ih, W_hh, b_ih, b_hh):
    batch = batch.astype(jnp.int32)
    batch3 = batch.reshape(NB, 1, BLK)
    idx = jnp.arange(NB)
    lo = batch[idx * BLK]
    hi = batch[idx * BLK + (BLK - 1)]
    w8 = jnp.minimum(lo // 8, (B - W) // 8)
    fast = (hi < w8 * 8 + W).astype(jnp.int32)
    win = jnp.stack([w8, fast], axis=1).astype(jnp.int32)
    wq = W_ih.T[:D] + W_hh.T
    wr = W_ih.T[D:]
    wqh, wql = _split(wq)
    wrh, wrl = _split(wr)
    bias = (b_ih + b_hh).reshape(1, 4 * D)

    grid_a = pltpu.PrefetchScalarGridSpec(
        num_scalar_prefetch=1,
        grid=(NB,),
        in_specs=[
            pl.BlockSpec((1, 1, BLK), lambda i, w: (i, 0, 0)),
            pl.BlockSpec((BLK, D), lambda i, w: (i, 0)),
            pl.BlockSpec((1, 4 * D), lambda i, w: (0, 0)),
        ],
        out_specs=[
            pl.BlockSpec((BLK, D), lambda i, w: (i, 0)),
            pl.BlockSpec((BLK, D), lambda i, w: (i, 0)),
            pl.BlockSpec((3 * B, D), lambda i, w: (0, 0)),
        ],
        scratch_shapes=[
            pltpu.VMEM((B, D), jnp.float32),
            pltpu.VMEM((B, 1), jnp.float32),
            pltpu.VMEM((B, 1), jnp.float32),
            pltpu.VMEM((B, D), jnp.float32),
        ],
    )
    xh, xl, st = pl.pallas_call(
        _body_a,
        grid_spec=grid_a,
        out_shape=(
            jax.ShapeDtypeStruct((N, D), jnp.bfloat16),
            jax.ShapeDtypeStruct((N, D), jnp.bfloat16),
            jax.ShapeDtypeStruct((3 * B, D), jnp.float32),
        ),
        compiler_params=pltpu.CompilerParams(
            dimension_semantics=("arbitrary",),
        ),
    )(win, batch3, x, bias)

    grid_b = pltpu.PrefetchScalarGridSpec(
        num_scalar_prefetch=1,
        grid=(T - 1, NB),
        in_specs=[
            pl.BlockSpec((1, 1, BLK), lambda t, i, w: (i, 0, 0)),
            pl.BlockSpec((BLK, D), lambda t, i, w: (i, 0)),
            pl.BlockSpec((BLK, D), lambda t, i, w: (i, 0)),
            pl.BlockSpec((D, 4 * D), lambda t, i, w: (0, 0)),
            pl.BlockSpec((D, 4 * D), lambda t, i, w: (0, 0)),
            pl.BlockSpec((D, 4 * D), lambda t, i, w: (0, 0)),
            pl.BlockSpec((D, 4 * D), lambda t, i, w: (0, 0)),
            pl.BlockSpec((1, 4 * D), lambda t, i, w: (0, 0)),
            pl.BlockSpec((3 * B, D), lambda t, i, w: (0, 0)),
        ],
        out_specs=pl.BlockSpec((B, 2 * D), lambda t, i, w: (0, 0)),
        scratch_shapes=[
            pltpu.VMEM((B, D), jnp.float32),
            pltpu.VMEM((B, D), jnp.float32),
            pltpu.VMEM((B, D), jnp.float32),
            pltpu.VMEM((B, 1), jnp.float32),
            pltpu.VMEM((B, 1), jnp.float32),
            pltpu.VMEM((B, D), jnp.float32),
        ],
    )
    return pl.pallas_call(
        _body_b,
        grid_spec=grid_b,
        out_shape=jax.ShapeDtypeStruct((B, 2 * D), jnp.float32),
        compiler_params=pltpu.CompilerParams(
            dimension_semantics=("arbitrary", "arbitrary"),
        ),
    )(win, batch3, xh, xl, wqh, wql, wrh, wrl, bias, st)


# speedup vs baseline: 1.0973x; 1.0973x over previous
import jax
import jax.numpy as jnp
from jax.experimental import pallas as pl
from jax.experimental.pallas import tpu as pltpu

D = 512
B = 64
T = 3
N = 50000
BLK = 5000
NB = N // BLK
NEG = -1e30
W = 16


def _split(a):
    hi = a.astype(jnp.bfloat16)
    lo = (a - hi.astype(jnp.float32)).astype(jnp.bfloat16)
    return hi, lo


def _update(w, wsz, seg, xh, xl, h_ref, m_ref, s_ref, u_ref):
    hw = h_ref[pl.ds(w, wsz), :]
    hwh, hwl = _split(hw)

    def dxt(u, v):
        return jax.lax.dot_general(u, v, (((1,), (1,)), ((), ())),
                                   preferred_element_type=jnp.float32)

    z = dxt(jnp.concatenate([hwh, hwl], axis=0), xh)
    e = z[:wsz] + z[wsz:] + dxt(hwh, xl)
    mask = (seg - w) == jax.lax.broadcasted_iota(jnp.int32, (wsz, BLK), 0)
    e_m = jnp.where(mask, e, NEG)
    m_old = m_ref[pl.ds(w, wsz), :]
    m_new = jnp.maximum(m_old, jnp.max(e_m, axis=1, keepdims=True))
    p = jnp.where(mask, jnp.exp(e - m_new), 0.0)
    scale = jnp.exp(m_old - m_new)
    s_ref[pl.ds(w, wsz), :] = (s_ref[pl.ds(w, wsz), :] * scale
                               + jnp.sum(p, axis=1, keepdims=True))
    ph, plo = _split(p)

    def dp(u, v):
        return jax.lax.dot_general(u, v, (((1,), (0,)), ((), ())),
                                   preferred_element_type=jnp.float32)

    y = dp(jnp.concatenate([ph, plo], axis=0), xh)
    u_ref[pl.ds(w, wsz), :] = (u_ref[pl.ds(w, wsz), :] * scale
                               + y[:wsz] + y[wsz:])
    m_ref[pl.ds(w, wsz), :] = m_new


def _softmax_blk(i, win_ref, seg, xh, xl, h_ref, m_ref, s_ref, u_ref):
    w8 = win_ref[i, 0]
    fast = win_ref[i, 1] == 1

    @pl.when(fast)
    def _fast():
        w = pl.multiple_of(w8 * 8, 8)
        _update(w, W, seg, xh, xl, h_ref, m_ref, s_ref, u_ref)

    @pl.when(jnp.logical_not(fast))
    def _general():
        _update(0, B, seg, xh, xl, h_ref, m_ref, s_ref, u_ref)


def _body_a(win_ref, batch_ref, x_ref, b_ref,
            xh_ref, xl_ref, st_ref,
            h_ref, m_ref, s_ref, u_ref):
    i = pl.program_id(0)

    @pl.when(i == 0)
    def _init():
        gates = jnp.broadcast_to(b_ref[...], (B, 4 * D))
        gi = jax.nn.sigmoid(gates[:, 0 * D:1 * D])
        gf = jax.nn.sigmoid(gates[:, 1 * D:2 * D])
        gg = jnp.tanh(gates[:, 2 * D:3 * D])
        go = jax.nn.sigmoid(gates[:, 3 * D:4 * D])
        c_new = gi * gg
        h_new = go * jnp.tanh(c_new)
        h_ref[...] = h_new
        st_ref[B:2 * B, :] = c_new
        m_ref[...] = jnp.full((B, 1), NEG, jnp.float32)
        s_ref[...] = jnp.zeros((B, 1), jnp.float32)
        u_ref[...] = jnp.zeros((B, D), jnp.float32)

    xh, xl = _split(x_ref[...])
    xh_ref[...] = xh
    xl_ref[...] = xl
    seg = batch_ref[0]
    _softmax_blk(i, win_ref, seg, xh, xl, h_ref, m_ref, s_ref, u_ref)

    @pl.when(i == NB - 1)
    def _finalize():
        st_ref[0:B, :] = h_ref[...]
        st_ref[2 * B:3 * B, :] = u_ref[...] / (s_ref[...] + 1e-16)


def _body_b(win_ref, batch_ref, xh_ref, xl_ref,
            wqh_ref, wql_ref, wrh_ref, wrl_ref, b_ref, st_ref,
            out_ref,
            h_ref, c_ref, r_ref, m_ref, s_ref, u_ref):
    t = pl.program_id(0)
    i = pl.program_id(1)

    @pl.when(i == 0)
    def _lstm_and_init():
        first = (t == 0)
        h_prev = jnp.where(first, st_ref[0:B, :], h_ref[...])
        c_prev = jnp.where(first, st_ref[B:2 * B, :], c_ref[...])
        r_prev = jnp.where(first, st_ref[2 * B:3 * B, :], r_ref[...])

        def d(u, v):
            return jax.lax.dot_general(u, v, (((1,), (0,)), ((), ())),
                                       preferred_element_type=jnp.float32)

        hh_, hl_ = _split(h_prev)
        rh_, rl_ = _split(r_prev)
        gates = (d(hh_, wqh_ref[...]) + d(hh_, wql_ref[...]) + d(hl_, wqh_ref[...])
                 + d(rh_, wrh_ref[...]) + d(rh_, wrl_ref[...]) + d(rl_, wrh_ref[...])
                 + b_ref[...])
        gi = jax.nn.sigmoid(gates[:, 0 * D:1 * D])
        gf = jax.nn.sigmoid(gates[:, 1 * D:2 * D])
        gg = jnp.tanh(gates[:, 2 * D:3 * D])
        go = jax.nn.sigmoid(gates[:, 3 * D:4 * D])
        c_new = gf * c_prev + gi * gg
        h_new = go * jnp.tanh(c_new)
        h_ref[...] = h_new
        c_ref[...] = c_new
        m_ref[...] = jnp.full((B, 1), NEG, jnp.float32)
        s_ref[...] = jnp.zeros((B, 1), jnp.float32)
        u_ref[...] = jnp.zeros((B, D), jnp.float32)

    seg = batch_ref[0]
    _softmax_blk(i, win_ref, seg, xh_ref[...], xl_ref[...],
                 h_ref, m_ref, s_ref, u_ref)

    @pl.when(i == NB - 1)
    def _finalize():
        r = u_ref[...] / (s_ref[...] + 1e-16)
        r_ref[...] = r

        @pl.when(t == T - 2)
        def _write_out():
            out_ref[:, :D] = h_ref[...]
            out_ref[:, D:] = r


def kernel(x, batch, W_ih, W_hh, b_ih, b_hh):
    batch = batch.astype(jnp.int32)
    batch3 = batch.reshape(NB, 1, BLK)
    idx = jnp.arange(NB)
    lo = batch[idx * BLK]
    hi = batch[idx * BLK + (BLK - 1)]
    w8 = jnp.minimum(lo // 8, (B - W) // 8)
    fast = (hi < w8 * 8 + W).astype(jnp.int32)
    win = jnp.stack([w8, fast], axis=1).astype(jnp.int32)
    wq = W_ih.T[:D] + W_hh.T
    wr = W_ih.T[D:]
    wqh, wql = _split(wq)
    wrh, wrl = _split(wr)
    bias = (b_ih + b_hh).reshape(1, 4 * D)

    grid_a = pltpu.PrefetchScalarGridSpec(
        num_scalar_prefetch=1,
        grid=(NB,),
        in_specs=[
            pl.BlockSpec((1, 1, BLK), lambda i, w: (i, 0, 0)),
            pl.BlockSpec((BLK, D), lambda i, w: (i, 0)),
            pl.BlockSpec((1, 4 * D), lambda i, w: (0, 0)),
        ],
        out_specs=[
            pl.BlockSpec((BLK, D), lambda i, w: (i, 0)),
            pl.BlockSpec((BLK, D), lambda i, w: (i, 0)),
            pl.BlockSpec((3 * B, D), lambda i, w: (0, 0)),
        ],
        scratch_shapes=[
            pltpu.VMEM((B, D), jnp.float32),
            pltpu.VMEM((B, 1), jnp.float32),
            pltpu.VMEM((B, 1), jnp.float32),
            pltpu.VMEM((B, D), jnp.float32),
        ],
    )
    xh, xl, st = pl.pallas_call(
        _body_a,
        grid_spec=grid_a,
        out_shape=(
            jax.ShapeDtypeStruct((N, D), jnp.bfloat16),
            jax.ShapeDtypeStruct((N, D), jnp.bfloat16),
            jax.ShapeDtypeStruct((3 * B, D), jnp.float32),
        ),
        compiler_params=pltpu.CompilerParams(
            dimension_semantics=("arbitrary",),
        ),
    )(win, batch3, x, bias)

    grid_b = pltpu.PrefetchScalarGridSpec(
        num_scalar_prefetch=1,
        grid=(T - 1, NB),
        in_specs=[
            pl.BlockSpec((1, 1, BLK), lambda t, i, w: (i, 0, 0)),
            pl.BlockSpec((BLK, D), lambda t, i, w: (i, 0)),
            pl.BlockSpec((BLK, D), lambda t, i, w: (i, 0)),
            pl.BlockSpec((D, 4 * D), lambda t, i, w: (0, 0)),
            pl.BlockSpec((D, 4 * D), lambda t, i, w: (0, 0)),
            pl.BlockSpec((D, 4 * D), lambda t, i, w: (0, 0)),
            pl.BlockSpec((D, 4 * D), lambda t, i, w: (0, 0)),
            pl.BlockSpec((1, 4 * D), lambda t, i, w: (0, 0)),
            pl.BlockSpec((3 * B, D), lambda t, i, w: (0, 0)),
        ],
        out_specs=pl.BlockSpec((B, 2 * D), lambda t, i, w: (0, 0)),
        scratch_shapes=[
            pltpu.VMEM((B, D), jnp.float32),
            pltpu.VMEM((B, D), jnp.float32),
            pltpu.VMEM((B, D), jnp.float32),
            pltpu.VMEM((B, 1), jnp.float32),
            pltpu.VMEM((B, 1), jnp.float32),
            pltpu.VMEM((B, D), jnp.float32),
        ],
    )
    return pl.pallas_call(
        _body_b,
        grid_spec=grid_b,
        out_shape=jax.ShapeDtypeStruct((B, 2 * D), jnp.float32),
        compiler_params=pltpu.CompilerParams(
            dimension_semantics=("arbitrary", "arbitrary"),
        ),
    )(win, batch3, xh, xl, wqh, wql, wrh, wrl, bias, st)
